# baseline (speedup 1.0000x reference)
import functools

import jax
import jax.numpy as jnp
from jax import lax
from jax.experimental import pallas as pl
from jax.experimental.pallas import tpu as pltpu
from jax.experimental.pallas import tpu_sc as plsc

EMBED = 64
SCALE = 8.0
NC, NS, L = 2, 16, 16
NW = NC * NS
W = 256
@functools.lru_cache(maxsize=None)
def _build(B: int, V: int, BATCH: int):
    b_per_w = B // NW
    nchunks = b_per_w // W
    rounds = nchunks // 2
    cps = BATCH // W
    mesh = plsc.VectorSubcoreMesh(core_axis_name="c", subcore_axis_name="s")

    @functools.partial(
        pl.kernel,
        mesh=mesh,
        out_type=jax.ShapeDtypeStruct((B // BATCH * EMBED, BATCH),
                                      jnp.float32),
        scratch_types=[
            [pltpu.VMEM((W,), jnp.int32) for _ in range(2)],
            [pltpu.VMEM((W,), jnp.int32) for _ in range(2)],
            [pltpu.VMEM((W,), jnp.int32) for _ in range(2)],
            [pltpu.VMEM((W, 2 * EMBED), jnp.float32) for _ in range(2)],
            [pltpu.VMEM((EMBED, W + 1), jnp.float32) for _ in range(2)],
            [pltpu.SemaphoreType.DMA for _ in range(2)],
            [pltpu.SemaphoreType.DMA for _ in range(2)],
            [pltpu.SemaphoreType.DMA for _ in range(2)],
        ],
        compiler_params=pltpu.CompilerParams(needs_layout_passes=False),
    )
    def emb(tok_hbm, table_hbm, out_hbm, tokb, idxb, hb, gbufs, tbufs,
            tsems, gsems, osems):
        wid = lax.axis_index("s") * NC + lax.axis_index("c")
        tbase = pl.multiple_of(wid * b_per_w, b_per_w)
        c0 = wid * nchunks
        jiota = [lax.iota(jnp.int32, L) + jg * L
                 for jg in range(EMBED // L)]

        def tok_start(c, b):
            pltpu.async_copy(
                tok_hbm.at[
                    pl.ds(pl.multiple_of(tbase + c * W, W), W)
                ],
                tokb[b], tsems[b],
            )

        def tok_wait(b):
            pltpu.make_async_copy(
                tok_hbm.at[pl.ds(tbase, W)], tokb[b], tsems[b]
            ).wait()

        def transform(b):
            @plsc.parallel_loop(0, W // L, 1, unroll=4)
            def _(i):
                sl = pl.ds(i * L, L)
                t = tokb[b][sl]
                idxb[b][sl] = lax.shift_right_logical(t, 1)
                hb[b][sl] = lax.shift_left(jnp.bitwise_and(t, 1), 6)

        def gather_start(b):
            pltpu.async_copy(table_hbm.at[idxb[b]], gbufs[b], gsems[b])

        def gather_wait(b):
            pltpu.make_async_copy(
                table_hbm.at[idxb[b]], gbufs[b], gsems[b]
            ).wait()

        def out_dst(c):
            cc = c0 + c
            s64 = pl.multiple_of((cc // cps) * EMBED, EMBED)
            b0 = pl.multiple_of((cc % cps) * W, W)
            return out_hbm.at[pl.ds(s64, EMBED), pl.ds(b0, W)]

        def out_start(c, b):
            pltpu.async_copy(tbufs[b].at[:, pl.ds(0, W)], out_dst(c), osems[b])

        def out_wait(b):
            pltpu.make_async_copy(
                tbufs[b].at[:, pl.ds(0, W)],
                out_hbm.at[pl.ds(0, EMBED), pl.ds(0, W)], osems[b]
            ).wait()

        def select(b):
            gbuf, tbuf = gbufs[b], tbufs[b]

            @plsc.parallel_loop(0, W // L, 1, unroll=4)
            def _(gi):
                r0 = gi * L
                hv = hb[b][pl.ds(r0, L)]
                for u in range(L):
                    h = hv[u]
                    r = r0 + u
                    grow = gbuf.at[r]
                    ridx = jnp.full((L,), r, jnp.int32)
                    for jg in range(EMBED // L):
                        vals = grow[pl.ds(h + jg * L, L)] * SCALE
                        plsc.store_scatter(tbuf, [jiota[jg], ridx], vals)

        tok_start(0, 0)
        tok_wait(0)
        transform(0)
        gather_start(0)
        tok_start(1, 1)

        def round_body(g, carry):
            not_last = g < rounds - 1
            for b in range(2):
                c = g * 2 + b
                nb = b ^ 1

                def prep_next():
                    tok_wait(nb)
                    transform(nb)
                    gather_start(nb)

                if b == 0:
                    prep_next()
                else:
                    pl.when(not_last)(prep_next)

                @pl.when(g > 0)
                def _():
                    out_wait(b)

                gather_wait(b)

                @pl.when(not_last)
                def _():
                    tok_start(c + 2, b)

                select(b)
                out_start(c, b)
            return carry

        lax.fori_loop(0, rounds, round_body, 0)
        out_wait(0)
        out_wait(1)

    return emb


def kernel(tokens, embedding_weight):
    BATCH, S = tokens.shape
    B = BATCH * S
    V = embedding_weight.shape[0]
    tokT = tokens.T.reshape(B).astype(jnp.int32)
    table2 = embedding_weight.reshape(V // 2, 2 * EMBED)
    out_n = _build(B, V, BATCH)(tokT, table2)
    return out_n.reshape(S, EMBED, BATCH).transpose(2, 0, 1)


# speedup vs baseline: 1.2037x; 1.2037x over previous
import functools

import jax
import jax.numpy as jnp
from jax import lax
from jax.experimental import pallas as pl
from jax.experimental.pallas import tpu as pltpu
from jax.experimental.pallas import tpu_sc as plsc

EMBED = 64
SCALE = 8.0
NC, NS, L = 2, 16, 16
NW = NC * NS
CHUNK = 320
NBUF = 2
RU = 8


@functools.lru_cache(maxsize=None)
def _build(B: int):
    b_per_w = B // NW
    nchunks = b_per_w // CHUNK
    rounds = nchunks // NBUF
    mesh = plsc.VectorSubcoreMesh(core_axis_name="c", subcore_axis_name="s")

    @functools.partial(
        pl.kernel,
        mesh=mesh,
        out_type=jax.ShapeDtypeStruct((B, EMBED), jnp.float32),
        scratch_types=[
            pltpu.VMEM((b_per_w,), jnp.int32),
            [pltpu.VMEM((CHUNK, EMBED), jnp.float32) for _ in range(NBUF)],
            [pltpu.VMEM((CHUNK, EMBED), jnp.float32) for _ in range(NBUF)],
            [pltpu.SemaphoreType.DMA for _ in range(NBUF)],
            [pltpu.SemaphoreType.DMA for _ in range(NBUF)],
        ],
        compiler_params=pltpu.CompilerParams(use_tc_tiling_on_sc=False),
    )
    def emb(tok_hbm, table_hbm, out_hbm, idx_v, gbufs, sbufs, gsems, ssems):
        wid = lax.axis_index("s") * NC + lax.axis_index("c")
        base = wid * b_per_w
        pltpu.sync_copy(tok_hbm.at[pl.ds(base, b_per_w)], idx_v)

        for b in range(NBUF):
            pltpu.async_copy(
                table_hbm.at[idx_v.at[pl.ds(b * CHUNK, CHUNK)]],
                gbufs[b], gsems[b],
            )

        def round_body(g, carry):
            for b in range(NBUF):
                off = (g * NBUF + b) * CHUNK
                gbuf, sbuf = gbufs[b], sbufs[b]
                pltpu.make_async_copy(
                    table_hbm.at[idx_v.at[pl.ds(off, CHUNK)]], gbuf, gsems[b]
                ).wait()

                @pl.when(g > 0)
                def _():
                    pltpu.make_async_copy(
                        sbuf, out_hbm.at[pl.ds(base, CHUNK)], ssems[b]
                    ).wait()

                def mul_body(i, c2):
                    r0 = i * RU
                    for u in range(RU):
                        for j in range(EMBED // L):
                            sl = pl.ds(j * L, L)
                            sbuf[r0 + u, sl] = gbuf[r0 + u, sl] * SCALE
                    return c2

                lax.fori_loop(0, CHUNK // RU, mul_body, 0)

                @pl.when(g < rounds - 1)
                def _():
                    pltpu.async_copy(
                        table_hbm.at[
                            idx_v.at[pl.ds(off + NBUF * CHUNK, CHUNK)]
                        ],
                        gbuf, gsems[b],
                    )

                pltpu.async_copy(
                    sbuf, out_hbm.at[pl.ds(base + off, CHUNK)], ssems[b]
                )
            return carry

        lax.fori_loop(0, rounds, round_body, 0)

        for b in range(NBUF):
            pltpu.make_async_copy(
                sbufs[b], out_hbm.at[pl.ds(base, CHUNK)], ssems[b]
            ).wait()

    return emb


def kernel(tokens, embedding_weight):
    B = tokens.shape[0] * tokens.shape[1]
    flat = tokens.reshape(B).astype(jnp.int32)
    out = _build(B)(flat, embedding_weight)
    return out.reshape(tokens.shape + (EMBED,))
